# Initial kernel scaffold; baseline (speedup 1.0000x reference)
import jax
import jax.numpy as jnp
from jax.experimental import pallas as pl


def kernel(atom_types, reordered_indices):
    raise NotImplementedError("write your pallas kernel here")


# speedup vs baseline: 3.6413x; 3.6413x over previous
import functools

import jax
import jax.numpy as jnp
from jax import lax
from jax.experimental import pallas as pl
from jax.experimental.pallas import tpu as pltpu
from jax.experimental.pallas import tpu_sc as plsc

_NUM_ROW = 13
_NUM_COL = 15
_WIDTH = _NUM_ROW + _NUM_COL
_WPAD = 32
_NTYPES = 118

_N = 819200
_NW = 32
_BPW = _N // _NW
_CHUNK = 1024
_NCHUNK = _BPW // _CHUNK


def _onehot_table_kernel(ri_ref, out_ref):
    ri = ri_ref[...]
    row = ri[:, 0:1]
    col = ri[:, 1:2]
    j = lax.broadcasted_iota(jnp.int32, (_NTYPES, _WPAD), 1)
    sel = jnp.clip(_NUM_ROW - j, 0, 1)
    target = row * sel + (col + _NUM_ROW) * (1 - sel)
    out_ref[...] = 1 - jnp.clip(jnp.abs(target - j), 0, 1)


def _build_lut(rolled_ri):
    return pl.pallas_call(
        _onehot_table_kernel,
        out_shape=jax.ShapeDtypeStruct((_NTYPES, _WPAD), jnp.int32),
    )(rolled_ri)


def _sc_gather_body(atoms_hbm, lut_hbm, out_hbm, idx_v, rows_v, sem0, sem1):
    wid = lax.axis_index("s") * 2 + lax.axis_index("c")
    base = wid * _BPW
    pltpu.sync_copy(atoms_hbm.at[pl.ds(base, _BPW)], idx_v)

    sems = (sem0, sem1)
    handles = {}

    def start(j):
        slot = j % 2
        handles[j] = pltpu.async_copy(
            lut_hbm.at[idx_v.at[pl.ds(j * _CHUNK, _CHUNK)]],
            rows_v.at[slot],
            sems[slot],
        )

    start(0)
    for j in range(_NCHUNK):
        if j + 1 < _NCHUNK:
            start(j + 1)
        handles[j].wait()
        pltpu.sync_copy(
            rows_v.at[j % 2], out_hbm.at[pl.ds(base + j * _CHUNK, _CHUNK)]
        )


def _sc_gather(atom_types, lut):
    mesh = plsc.VectorSubcoreMesh(core_axis_name="c", subcore_axis_name="s")
    run = functools.partial(
        pl.kernel,
        mesh=mesh,
        out_type=jax.ShapeDtypeStruct((_N, _WPAD), jnp.int32),
        scratch_types=[
            pltpu.VMEM((_BPW,), jnp.int32),
            pltpu.VMEM((2, _CHUNK, _WPAD), jnp.int32),
            pltpu.SemaphoreType.DMA,
            pltpu.SemaphoreType.DMA,
        ],
        compiler_params=pltpu.CompilerParams(use_tc_tiling_on_sc=False),
    )(_sc_gather_body)
    return run(atom_types, lut)


def kernel(atom_types, reordered_indices):
    rolled = jnp.roll(reordered_indices, 1, axis=0)
    lut = _build_lut(rolled)
    out32 = _sc_gather(atom_types, lut)
    return out32[:, :_WIDTH]


# speedup vs baseline: 4.0982x; 1.1255x over previous
import functools

import jax
import jax.numpy as jnp
from jax import lax
from jax.experimental import pallas as pl
from jax.experimental.pallas import tpu as pltpu
from jax.experimental.pallas import tpu_sc as plsc

_NUM_ROW = 13
_NUM_COL = 15
_WIDTH = _NUM_ROW + _NUM_COL
_NTYPES = 118
_LUTPAD = 128

_N = 819200
_NW = 32
_BPW = _N // _NW
_LANES = 16
_CHUNK = 1024
_NCHUNK = _BPW // _CHUNK
_GROUPS = _CHUNK // _LANES


def _sc_body(atoms_hbm, mask_hbm, out_hbm, idx_v, lut_v, obuf_v, sem0, sem1):
    wid = lax.axis_index("s") * 2 + lax.axis_index("c")
    base = wid * _BPW
    pltpu.sync_copy(mask_hbm, lut_v)
    pltpu.sync_copy(atoms_hbm.at[pl.ds(base, _BPW)], idx_v)

    lane = lax.iota(jnp.int32, _LANES)
    sems = (sem0, sem1)
    handles = {}

    def fill(c, slot):
        def group(g, _):
            a0 = pl.multiple_of(c * _CHUNK + g * _LANES, _LANES)
            atoms = idx_v[pl.ds(a0, _LANES)]
            w = plsc.load_gather(lut_v, [atoms])
            rloc = pl.multiple_of(g * _LANES, _LANES) + lane
            for j in range(_WIDTH):
                bit = lax.shift_right_logical(w, j) & 1
                plsc.store_scatter(
                    obuf_v.at[slot], [rloc, jnp.full((_LANES,), j, jnp.int32)], bit
                )
            return 0

        lax.fori_loop(0, _GROUPS, group, 0)

    def flush(c, slot):
        handles[c] = pltpu.async_copy(
            obuf_v.at[slot],
            out_hbm.at[pl.ds(base + c * _CHUNK, _CHUNK)],
            sems[slot],
        )

    for c in range(_NCHUNK):
        slot = c % 2
        if c >= 2:
            handles[c - 2].wait()
        fill(c, slot)
        flush(c, slot)
    handles[_NCHUNK - 2].wait()
    handles[_NCHUNK - 1].wait()


def _sc_encode(atom_types, mask_lut):
    mesh = plsc.VectorSubcoreMesh(core_axis_name="c", subcore_axis_name="s")
    run = functools.partial(
        pl.kernel,
        mesh=mesh,
        out_type=jax.ShapeDtypeStruct((_N, _WIDTH), jnp.int32),
        scratch_types=[
            pltpu.VMEM((_BPW,), jnp.int32),
            pltpu.VMEM((_LUTPAD,), jnp.int32),
            pltpu.VMEM((2, _CHUNK, _WIDTH), jnp.int32),
            pltpu.SemaphoreType.DMA,
            pltpu.SemaphoreType.DMA,
        ],
        compiler_params=pltpu.CompilerParams(
            use_tc_tiling_on_sc=False, needs_layout_passes=False
        ),
    )(_sc_body)
    return run(atom_types, mask_lut)


def kernel(atom_types, reordered_indices):
    rolled = jnp.roll(reordered_indices, 1, axis=0)
    mask = (1 << rolled[:, 0]) | (1 << (_NUM_ROW + rolled[:, 1]))
    mask = jnp.zeros((_LUTPAD,), jnp.int32).at[:_NTYPES].set(mask.astype(jnp.int32))
    return _sc_encode(atom_types, mask)


# speedup vs baseline: 4.1621x; 1.0156x over previous
import functools

import jax
import jax.numpy as jnp
from jax import lax
from jax.experimental import pallas as pl
from jax.experimental.pallas import tpu as pltpu
from jax.experimental.pallas import tpu_sc as plsc

_NUM_ROW = 13
_NUM_COL = 15
_WIDTH = _NUM_ROW + _NUM_COL
_NTYPES = 118
_LUTPAD = 128

_N = 819200
_NW = 32
_BPW = _N // _NW
_LANES = 16
_CHUNK = 1280
_NCHUNK = _BPW // _CHUNK
_GROUPS = _CHUNK // _LANES


def _sc_body(atoms_hbm, mask_hbm, out_hbm, idx_v, lut_v, obuf_v, sem0, sem1):
    wid = lax.axis_index("s") * 2 + lax.axis_index("c")
    base = wid * _BPW
    pltpu.sync_copy(mask_hbm, lut_v)
    pltpu.sync_copy(atoms_hbm.at[pl.ds(base, _BPW)], idx_v)

    lane = lax.iota(jnp.int32, _LANES)
    jcol = [jnp.full((_LANES,), j, jnp.int32) for j in range(_WIDTH)]
    sems = (sem0, sem1)

    def fill(c, slot):
        @plsc.parallel_loop(0, _GROUPS, unroll=4)
        def group(g):
            a0 = pl.multiple_of(c * _CHUNK + g * _LANES, _LANES)
            atoms = idx_v[pl.ds(a0, _LANES)]
            w = plsc.load_gather(lut_v, [atoms])
            rloc = g * _LANES + lane
            for j in range(_WIDTH):
                bit = lax.shift_right_logical(w, j) & 1
                plsc.store_scatter(obuf_v.at[slot], [rloc, jcol[j]], bit)

    def flush(c, slot):
        r0 = pl.multiple_of(base + c * _CHUNK, _CHUNK)
        pltpu.async_copy(obuf_v.at[slot], out_hbm.at[pl.ds(r0, _CHUNK)], sems[slot])

    def drain(slot):
        pltpu.make_async_copy(
            out_hbm.at[pl.ds(base, _CHUNK)], obuf_v.at[slot], sems[slot]
        ).wait()

    def super_step(cs, _):
        for slot in range(2):
            c = cs * 2 + slot

            @pl.when(cs > 0)
            def _():
                drain(slot)

            fill(c, slot)
            flush(c, slot)
        return 0

    lax.fori_loop(0, _NCHUNK // 2, super_step, 0)
    drain(0)
    drain(1)


def _sc_encode(atom_types, mask_lut):
    mesh = plsc.VectorSubcoreMesh(core_axis_name="c", subcore_axis_name="s")
    run = functools.partial(
        pl.kernel,
        mesh=mesh,
        out_type=jax.ShapeDtypeStruct((_N, _WIDTH), jnp.int32),
        scratch_types=[
            pltpu.VMEM((_BPW,), jnp.int32),
            pltpu.VMEM((_LUTPAD,), jnp.int32),
            pltpu.VMEM((2, _CHUNK, _WIDTH), jnp.int32),
            pltpu.SemaphoreType.DMA,
            pltpu.SemaphoreType.DMA,
        ],
        compiler_params=pltpu.CompilerParams(
            use_tc_tiling_on_sc=False, needs_layout_passes=False
        ),
    )(_sc_body)
    return run(atom_types, mask_lut)


def kernel(atom_types, reordered_indices):
    rolled = jnp.roll(reordered_indices, 1, axis=0)
    mask = (1 << rolled[:, 0]) | (1 << (_NUM_ROW + rolled[:, 1]))
    mask = jnp.zeros((_LUTPAD,), jnp.int32).at[:_NTYPES].set(mask.astype(jnp.int32))
    return _sc_encode(atom_types, mask)


# speedup vs baseline: 6.5377x; 1.5708x over previous
import functools

import jax
import jax.numpy as jnp
from jax import lax
from jax.experimental import pallas as pl
from jax.experimental.pallas import tpu as pltpu
from jax.experimental.pallas import tpu_sc as plsc

_NUM_ROW = 13
_NUM_COL = 15
_WIDTH = _NUM_ROW + _NUM_COL
_NTYPES = 118
_LUTPAD = 128

_N = 819200
_NW = 32
_BPW = _N // _NW
_LANES = 16
_CHUNK = 1280
_NCHUNK = _BPW // _CHUNK
_GROUPS = _CHUNK // _LANES


def _sc_body(atoms_hbm, mask_hbm, out_hbm, idx_v, lut_v, obuf_v, sem0, sem1):
    wid = lax.axis_index("s") * 2 + lax.axis_index("c")
    base = wid * _BPW
    pltpu.sync_copy(mask_hbm, lut_v)
    pltpu.sync_copy(atoms_hbm.at[pl.ds(base, _BPW)], idx_v)

    lane = lax.iota(jnp.int32, _LANES)
    lane12 = lane + 12
    sems = (sem0, sem1)

    def fill(c, slot):
        @plsc.parallel_loop(0, _GROUPS, unroll=2)
        def group(g):
            a0 = pl.multiple_of(c * _CHUNK + g * _LANES, _LANES)
            atoms = idx_v[pl.ds(a0, _LANES)]
            w = plsc.load_gather(lut_v, [atoms])
            r0 = pl.multiple_of(g * _LANES, _LANES)
            for i in range(_LANES):
                wb = w.at[jnp.full((_LANES,), i, jnp.int32)].get(
                    mode="promise_in_bounds"
                )
                lo = lax.shift_right_logical(wb, lane) & 1
                hi = lax.shift_right_logical(wb, lane12) & 1
                obuf_v[slot, r0 + i, pl.ds(0, _LANES)] = lo
                obuf_v[slot, r0 + i, pl.ds(12, _LANES)] = hi

    def flush(c, slot):
        r0 = pl.multiple_of(base + c * _CHUNK, _CHUNK)
        pltpu.async_copy(obuf_v.at[slot], out_hbm.at[pl.ds(r0, _CHUNK)], sems[slot])

    def drain(slot):
        pltpu.make_async_copy(
            out_hbm.at[pl.ds(base, _CHUNK)], obuf_v.at[slot], sems[slot]
        ).wait()

    def super_step(cs, _):
        for slot in range(2):
            c = cs * 2 + slot

            @pl.when(cs > 0)
            def _():
                drain(slot)

            fill(c, slot)
            flush(c, slot)
        return 0

    lax.fori_loop(0, _NCHUNK // 2, super_step, 0)
    drain(0)
    drain(1)


def _sc_encode(atom_types, mask_lut):
    mesh = plsc.VectorSubcoreMesh(core_axis_name="c", subcore_axis_name="s")
    run = functools.partial(
        pl.kernel,
        mesh=mesh,
        out_type=jax.ShapeDtypeStruct((_N, _WIDTH), jnp.int32),
        scratch_types=[
            pltpu.VMEM((_BPW,), jnp.int32),
            pltpu.VMEM((_LUTPAD,), jnp.int32),
            pltpu.VMEM((2, _CHUNK, _WIDTH), jnp.int32),
            pltpu.SemaphoreType.DMA,
            pltpu.SemaphoreType.DMA,
        ],
        compiler_params=pltpu.CompilerParams(
            use_tc_tiling_on_sc=False, needs_layout_passes=False
        ),
    )(_sc_body)
    return run(atom_types, mask_lut)


def kernel(atom_types, reordered_indices):
    rolled = jnp.roll(reordered_indices, 1, axis=0)
    mask = (1 << rolled[:, 0]) | (1 << (_NUM_ROW + rolled[:, 1]))
    mask = jnp.zeros((_LUTPAD,), jnp.int32).at[:_NTYPES].set(mask.astype(jnp.int32))
    return _sc_encode(atom_types, mask)
